# baseline (speedup 1.0000x reference)
import functools

import jax
import jax.numpy as jnp
from jax import lax
from jax.experimental import pallas as pl
from jax.experimental.pallas import tpu as pltpu
from jax.experimental.pallas import tpu_sc as plsc

EPS = 1e-05
LN_EPS = 1e-05
CAPACITY_FACTOR = 1.0


def kernel(hidden_states, ln_scale, ln_bias, gate_w, gate_proj_w, up_proj_w, down_proj_w):
    B, S, H = hidden_states.shape
    hid = hidden_states.reshape(-1, H)
    N = hid.shape[0]
    E = gate_w.shape[0]
    I = gate_proj_w.shape[1]
    C = int(N * CAPACITY_FACTOR / E)
    C = max(C, 1)
    C = min(C, N)

    def _router_kernel(x_ref, gw_ref, scale_ref, bias_ref,
                       prob_ref, aux_ref):
        x = x_ref[...]
        mean = jnp.mean(x, axis=1, keepdims=True)
        xc = x - mean
        var = jnp.mean(xc * xc, axis=1, keepdims=True)
        xn = xc * jax.lax.rsqrt(var + LN_EPS) * scale_ref[...] + bias_ref[...]
        logits = jax.lax.dot_general(
            gw_ref[...], xn, (((1,), (1,)), ((), ())),
            preferred_element_type=jnp.float32)

        tokmax = jnp.max(logits, axis=1, keepdims=True)
        ex = jnp.exp(logits - tokmax)
        denom = jnp.sum(ex, axis=1, keepdims=True)
        pfull = ex / denom

        expmax = jnp.max(logits, axis=0, keepdims=True)
        lse = jnp.log(jnp.sum(jnp.exp(logits - expmax), axis=0,
                              keepdims=True)) + expmax
        aux_ref[...] = jnp.full((1, 1), 0.001, jnp.float32) * jnp.mean(lse * lse)

        prob_ref[...] = pfull

    pfull_t, aux = pl.pallas_call(
        _router_kernel,
        out_shape=[
            jax.ShapeDtypeStruct((E, N), jnp.float32),
            jax.ShapeDtypeStruct((1, 1), jnp.float32),
        ],
    )(hid, gate_w, ln_scale.reshape(1, H), ln_bias.reshape(1, H))

    info = plsc.get_sparse_core_info()
    NW = info.num_cores * info.num_subcores
    EPW = E // NW
    LANES = info.num_lanes
    NBLK = LANES
    BLK = N // NBLK
    CPB = BLK // LANES
    mesh = plsc.VectorSubcoreMesh(core_axis_name="c", subcore_axis_name="s")

    @functools.partial(
        pl.kernel, mesh=mesh,
        out_type=[
            jax.ShapeDtypeStruct((E * C,), jnp.int32),
            jax.ShapeDtypeStruct((E * C,), jnp.float32),
            jax.ShapeDtypeStruct((E * C, H), jnp.float32),
        ],
        scratch_types=[
            pltpu.VMEM((N,), jnp.float32),
            pltpu.VMEM((C,), jnp.int32),
            pltpu.VMEM((C,), jnp.float32),
            pltpu.VMEM((C, H), jnp.float32),
            pltpu.SemaphoreType.DMA,
        ],
    )
    def _sc_route(probf_hbm, hid_hbm, idx_hbm, p_hbm, xg_hbm,
                  pv, idxo, probo, xrows, sem):
        wid = lax.axis_index("s") * info.num_cores + lax.axis_index("c")
        lane = lax.iota(jnp.int32, LANES)

        def allmax(v):
            for sh in (8, 4, 2, 1):
                v = jnp.maximum(v, v[jnp.bitwise_xor(lane, sh)])
            return v

        def allmin_i(v):
            for sh in (8, 4, 2, 1):
                v = jnp.minimum(v, v[jnp.bitwise_xor(lane, sh)])
            return v

        for ex in range(EPW):
            e = wid * EPW + ex
            pltpu.sync_copy(probf_hbm.at[pl.ds(e * N, N)], pv)

            bmax = jnp.full((LANES,), -1.0, jnp.float32)
            for b in range(NBLK):
                lm = pv[pl.ds(b * BLK, LANES)]
                for m in range(1, CPB):
                    lm = jnp.maximum(lm, pv[pl.ds(b * BLK + m * LANES, LANES)])
                bmax = jnp.where(lane == b, allmax(lm), bmax)

            def k_body(k, carry):
                bmax, oi_lo, oi_hi, ov_lo, ov_hi = carry
                sv = allmax(bmax)
                bc = jnp.where(bmax == sv, lane, jnp.full((LANES,), NBLK, jnp.int32))
                b0v = allmin_i(bc)
                b0 = b0v[0]
                base = b0 * BLK
                cand = jnp.full((LANES,), BLK, jnp.int32)
                for m in range(CPB):
                    v = pv[pl.ds(base + m * LANES, LANES)]
                    cand = jnp.minimum(
                        cand, jnp.where(v == sv,
                                        jnp.full((LANES,), m * LANES, jnp.int32) + lane,
                                        jnp.full((LANES,), BLK, jnp.int32)))
                posv = allmin_i(cand)
                fv = b0v * BLK + posv
                at_lo = lane == k
                at_hi = lane == (k - LANES)
                oi_lo = jnp.where(at_lo, fv, oi_lo)
                oi_hi = jnp.where(at_hi, fv, oi_hi)
                ov_lo = jnp.where(at_lo, sv, ov_lo)
                ov_hi = jnp.where(at_hi, sv, ov_hi)
                jc = lax.shift_right_logical(fv[0], 4)
                w = pv[pl.ds(jc * LANES, LANES)]
                w = jnp.where(lane == jnp.bitwise_and(fv, LANES - 1),
                              jnp.full((LANES,), -1.0, jnp.float32), w)
                pv[pl.ds(jc * LANES, LANES)] = w
                nm = pv[pl.ds(base, LANES)]
                for m in range(1, CPB):
                    nm = jnp.maximum(nm, pv[pl.ds(base + m * LANES, LANES)])
                bmax = jnp.where(lane == b0, allmax(nm), bmax)
                return (bmax, oi_lo, oi_hi, ov_lo, ov_hi)

            zi = jnp.zeros((LANES,), jnp.int32)
            zf = jnp.zeros((LANES,), jnp.float32)
            _, oi_lo, oi_hi, ov_lo, ov_hi = lax.fori_loop(
                0, C, k_body, (bmax, zi, zi, zf, zf))
            idxo[pl.ds(0, LANES)] = oi_lo
            idxo[pl.ds(LANES, LANES)] = oi_hi
            probo[pl.ds(0, LANES)] = ov_lo
            probo[pl.ds(LANES, LANES)] = ov_hi

            pltpu.async_copy(hid_hbm.at[idxo], xrows, sem).wait()
            pltpu.sync_copy(idxo, idx_hbm.at[pl.ds(e * C, C)])
            pltpu.sync_copy(probo, p_hbm.at[pl.ds(e * C, C)])
            pltpu.sync_copy(xrows, xg_hbm.at[pl.ds(e * C, C)])

    idx_f, prob_f, xgath = _sc_route(pfull_t.reshape(-1), hid)
    idx = idx_f.reshape(E, C)
    prob = prob_f.reshape(E, C)

    def _ffn_kernel(idx_ref, prob_ref, xin_ref, gp_ref, up_ref, dp_ref,
                    out_ref, cnt_ref):
        e = pl.program_id(0)

        @pl.when(e == 0)
        def _():
            out_ref[...] = jnp.zeros_like(out_ref)
            cnt_ref[...] = jnp.zeros_like(cnt_ref)

        x = xin_ref[...]
        g = jax.lax.dot_general(x, gp_ref[0], (((1,), (1,)), ((), ())),
                                preferred_element_type=jnp.float32)
        u = jax.lax.dot_general(x, up_ref[0], (((1,), (1,)), ((), ())),
                                preferred_element_type=jnp.float32)
        h = g * jax.nn.sigmoid(g) * u
        o = jax.lax.dot_general(h, dp_ref[0], (((1,), (1,)), ((), ())),
                                preferred_element_type=jnp.float32)
        for c in range(C):
            t = idx_ref[e, c]
            p = prob_ref[e, c]
            out_ref[t, :] = out_ref[t, :] + o[c, :] * p
            cnt_ref[pl.ds(t, 1), :] = cnt_ref[pl.ds(t, 1), :] + p

        @pl.when(e == pl.num_programs(0) - 1)
        def _():
            out_ref[...] = out_ref[...] / jnp.maximum(cnt_ref[...], EPS)

    out = pl.pallas_call(
        _ffn_kernel,
        grid=(E,),
        in_specs=[
            pl.BlockSpec(memory_space=pltpu.SMEM),
            pl.BlockSpec(memory_space=pltpu.SMEM),
            pl.BlockSpec((C, H), lambda e: (e, 0)),
            pl.BlockSpec((1, I, H), lambda e: (e, 0, 0)),
            pl.BlockSpec((1, I, H), lambda e: (e, 0, 0)),
            pl.BlockSpec((1, H, I), lambda e: (e, 0, 0)),
        ],
        out_specs=pl.BlockSpec((N, H), lambda e: (0, 0)),
        out_shape=jax.ShapeDtypeStruct((N, H), jnp.float32),
        scratch_shapes=[pltpu.VMEM((N, 1), jnp.float32)],
        compiler_params=pltpu.CompilerParams(
            dimension_semantics=("arbitrary",)),
    )(idx, prob, xgath.reshape(E * C, H), gate_proj_w, up_proj_w, down_proj_w)

    return out.reshape(B, S, H), aux.reshape(())


# speedup vs baseline: 1.0072x; 1.0072x over previous
import functools

import jax
import jax.numpy as jnp
from jax import lax
from jax.experimental import pallas as pl
from jax.experimental.pallas import tpu as pltpu
from jax.experimental.pallas import tpu_sc as plsc

EPS = 1e-05
LN_EPS = 1e-05
CAPACITY_FACTOR = 1.0


def kernel(hidden_states, ln_scale, ln_bias, gate_w, gate_proj_w, up_proj_w, down_proj_w):
    B, S, H = hidden_states.shape
    hid = hidden_states.reshape(-1, H)
    N = hid.shape[0]
    E = gate_w.shape[0]
    I = gate_proj_w.shape[1]
    C = int(N * CAPACITY_FACTOR / E)
    C = max(C, 1)
    C = min(C, N)

    def _router_kernel(x_ref, gw_ref, scale_ref, bias_ref,
                       prob_ref, aux_ref):
        x = x_ref[...]
        mean = jnp.mean(x, axis=1, keepdims=True)
        xc = x - mean
        var = jnp.mean(xc * xc, axis=1, keepdims=True)
        xn = xc * jax.lax.rsqrt(var + LN_EPS) * scale_ref[...] + bias_ref[...]
        logits = jax.lax.dot_general(
            gw_ref[...], xn, (((1,), (1,)), ((), ())),
            preferred_element_type=jnp.float32)

        tokmax = jnp.max(logits, axis=1, keepdims=True)
        ex = jnp.exp(logits - tokmax)
        denom = jnp.sum(ex, axis=1, keepdims=True)
        pfull = ex / denom

        expmax = jnp.max(logits, axis=0, keepdims=True)
        lse = jnp.log(jnp.sum(jnp.exp(logits - expmax), axis=0,
                              keepdims=True)) + expmax
        aux_ref[...] = jnp.full((1, 1), 0.001, jnp.float32) * jnp.mean(lse * lse)

        prob_ref[...] = pfull

    pfull_t, aux = pl.pallas_call(
        _router_kernel,
        out_shape=[
            jax.ShapeDtypeStruct((E, N), jnp.float32),
            jax.ShapeDtypeStruct((1, 1), jnp.float32),
        ],
    )(hid, gate_w, ln_scale.reshape(1, H), ln_bias.reshape(1, H))

    info = plsc.get_sparse_core_info()
    NW = info.num_cores * info.num_subcores
    EPW = E // NW
    LANES = info.num_lanes
    NBLK = LANES
    BLK = N // NBLK
    CPB = BLK // LANES
    mesh = plsc.VectorSubcoreMesh(core_axis_name="c", subcore_axis_name="s")

    @functools.partial(
        pl.kernel, mesh=mesh,
        out_type=[
            jax.ShapeDtypeStruct((E * C,), jnp.int32),
            jax.ShapeDtypeStruct((E * C,), jnp.float32),
            jax.ShapeDtypeStruct((E * C, H), jnp.float32),
        ],
        scratch_types=[
            pltpu.VMEM((EPW, N), jnp.float32),
            pltpu.VMEM((EPW, C), jnp.int32),
            pltpu.VMEM((EPW, C), jnp.float32),
            pltpu.VMEM((EPW, C, H), jnp.float32),
            pltpu.SemaphoreType.DMA,
            pltpu.SemaphoreType.DMA,
            pltpu.SemaphoreType.DMA,
            pltpu.SemaphoreType.DMA,
        ],
    )
    def _sc_route(probf_hbm, hid_hbm, idx_hbm, p_hbm, xg_hbm,
                  pvs, idxos, probos, xrs, semp0, semp1, semg0, semg1):
        wid = lax.axis_index("s") * info.num_cores + lax.axis_index("c")
        lane = lax.iota(jnp.int32, LANES)
        psems = [semp0, semp1]
        gsems = [semg0, semg1]

        def allmax(v):
            for sh in (8, 4, 2, 1):
                v = jnp.maximum(v, v[jnp.bitwise_xor(lane, sh)])
            return v

        def allmin_i(v):
            for sh in (8, 4, 2, 1):
                v = jnp.minimum(v, v[jnp.bitwise_xor(lane, sh)])
            return v

        def topk(pv, idxo, probo):
            bmax = jnp.full((LANES,), -1.0, jnp.float32)
            for b in range(NBLK):
                lm = pv[pl.ds(b * BLK, LANES)]
                for m in range(1, CPB):
                    lm = jnp.maximum(lm, pv[pl.ds(b * BLK + m * LANES, LANES)])
                bmax = jnp.where(lane == b, allmax(lm), bmax)

            def k_body(k, carry):
                bmax, oi_lo, oi_hi, ov_lo, ov_hi = carry
                sv = allmax(bmax)
                bc = jnp.where(bmax == sv, lane, jnp.full((LANES,), NBLK, jnp.int32))
                b0v = allmin_i(bc)
                b0 = b0v[0]
                base = b0 * BLK
                cand = jnp.full((LANES,), BLK, jnp.int32)
                for m in range(CPB):
                    v = pv[pl.ds(base + m * LANES, LANES)]
                    cand = jnp.minimum(
                        cand, jnp.where(v == sv,
                                        jnp.full((LANES,), m * LANES, jnp.int32) + lane,
                                        jnp.full((LANES,), BLK, jnp.int32)))
                posv = allmin_i(cand)
                fv = b0v * BLK + posv
                at_lo = lane == k
                at_hi = lane == (k - LANES)
                oi_lo = jnp.where(at_lo, fv, oi_lo)
                oi_hi = jnp.where(at_hi, fv, oi_hi)
                ov_lo = jnp.where(at_lo, sv, ov_lo)
                ov_hi = jnp.where(at_hi, sv, ov_hi)
                jc = lax.shift_right_logical(fv[0], 4)
                w = pv[pl.ds(jc * LANES, LANES)]
                w = jnp.where(lane == jnp.bitwise_and(fv, LANES - 1),
                              jnp.full((LANES,), -1.0, jnp.float32), w)
                pv[pl.ds(jc * LANES, LANES)] = w
                nm = pv[pl.ds(base, LANES)]
                for m in range(1, CPB):
                    nm = jnp.maximum(nm, pv[pl.ds(base + m * LANES, LANES)])
                bmax = jnp.where(lane == b0, allmax(nm), bmax)
                return (bmax, oi_lo, oi_hi, ov_lo, ov_hi)

            zi = jnp.zeros((LANES,), jnp.int32)
            zf = jnp.zeros((LANES,), jnp.float32)
            _, oi_lo, oi_hi, ov_lo, ov_hi = lax.fori_loop(
                0, C, k_body, (bmax, zi, zi, zf, zf))
            idxo[pl.ds(0, LANES)] = oi_lo
            idxo[pl.ds(LANES, LANES)] = oi_hi
            probo[pl.ds(0, LANES)] = ov_lo
            probo[pl.ds(LANES, LANES)] = ov_hi

        pcopies = []
        for ex in range(EPW):
            e = wid * EPW + ex
            pcopies.append(pltpu.async_copy(
                probf_hbm.at[pl.ds(e * N, N)], pvs.at[ex], psems[ex]))
        gcopies = []
        for ex in range(EPW):
            pcopies[ex].wait()
            topk(pvs.at[ex], idxos.at[ex], probos.at[ex])
            gcopies.append(pltpu.async_copy(
                hid_hbm.at[idxos.at[ex]], xrs.at[ex], gsems[ex]))
        for ex in range(EPW):
            e = wid * EPW + ex
            gcopies[ex].wait()
            pltpu.sync_copy(idxos.at[ex], idx_hbm.at[pl.ds(e * C, C)])
            pltpu.sync_copy(probos.at[ex], p_hbm.at[pl.ds(e * C, C)])
            pltpu.sync_copy(xrs.at[ex], xg_hbm.at[pl.ds(e * C, C)])

    idx_f, prob_f, xgath = _sc_route(pfull_t.reshape(-1), hid)
    idx = idx_f.reshape(E, C)
    prob = prob_f.reshape(E, C)

    def _ffn_kernel(idx_ref, prob_ref, xin_ref, gp_ref, up_ref, dp_ref,
                    out_ref, cnt_ref):
        e = pl.program_id(0)

        @pl.when(e == 0)
        def _():
            out_ref[...] = jnp.zeros_like(out_ref)
            cnt_ref[...] = jnp.zeros_like(cnt_ref)

        x = xin_ref[...]
        g = jax.lax.dot_general(x, gp_ref[0], (((1,), (1,)), ((), ())),
                                preferred_element_type=jnp.float32)
        u = jax.lax.dot_general(x, up_ref[0], (((1,), (1,)), ((), ())),
                                preferred_element_type=jnp.float32)
        h = g * jax.nn.sigmoid(g) * u
        o = jax.lax.dot_general(h, dp_ref[0], (((1,), (1,)), ((), ())),
                                preferred_element_type=jnp.float32)
        for c in range(C):
            t = idx_ref[e, c]
            p = prob_ref[e, c]
            out_ref[t, :] = out_ref[t, :] + o[c, :] * p
            cnt_ref[pl.ds(t, 1), :] = cnt_ref[pl.ds(t, 1), :] + p

        @pl.when(e == pl.num_programs(0) - 1)
        def _():
            out_ref[...] = out_ref[...] / jnp.maximum(cnt_ref[...], EPS)

    out = pl.pallas_call(
        _ffn_kernel,
        grid=(E,),
        in_specs=[
            pl.BlockSpec(memory_space=pltpu.SMEM),
            pl.BlockSpec(memory_space=pltpu.SMEM),
            pl.BlockSpec((C, H), lambda e: (e, 0)),
            pl.BlockSpec((1, I, H), lambda e: (e, 0, 0)),
            pl.BlockSpec((1, I, H), lambda e: (e, 0, 0)),
            pl.BlockSpec((1, H, I), lambda e: (e, 0, 0)),
        ],
        out_specs=pl.BlockSpec((N, H), lambda e: (0, 0)),
        out_shape=jax.ShapeDtypeStruct((N, H), jnp.float32),
        scratch_shapes=[pltpu.VMEM((N, 1), jnp.float32)],
        compiler_params=pltpu.CompilerParams(
            dimension_semantics=("arbitrary",)),
    )(idx, prob, xgath.reshape(E * C, H), gate_proj_w, up_proj_w, down_proj_w)

    return out.reshape(B, S, H), aux.reshape(())
